# baseline (speedup 1.0000x reference)
import jax
import jax.numpy as jnp
from jax import lax
from jax.experimental import pallas as pl
from jax.experimental.pallas import tpu as pltpu
from jax.experimental.pallas import tpu_sc as plsc

LV_LR = 0.01
SIGMA = 1.0
NOISE_SCALE = (2.0 * LV_LR) ** 0.5

NC = 2
NS = 16
NW = NC * NS
L = 16


def _update_body(lv_ref, h_ref, d_ref, nz_ref, w_ref, b_ref, out_ref):
    hb = h_ref[...]
    lv = lv_ref[:, :64] * (1.0 - hb) + lv_ref[:, 64:] * hb
    w = w_ref[...]
    pred = jnp.dot(lv, w, preferred_element_type=jnp.float32) + b_ref[...]
    resid = d_ref[...] - pred
    g = lax.dot_general(
        resid, w, (((1,), (1,)), ((), ())), preferred_element_type=jnp.float32
    ) - lv
    out_ref[:, :64] = LV_LR * g + NOISE_SCALE * nz_ref[:, 0, 0, :]
    out_ref[:, 64:] = jnp.zeros_like(g)


def _compute_update(lv128, h2, data, noise4, W, b, P, B, D, DD):
    TB = 1024
    nj = B // TB
    return pl.pallas_call(
        _update_body,
        grid=(nj, P),
        in_specs=[
            pl.BlockSpec((TB, 2 * D), lambda j, p: (p * nj + j, 0)),
            pl.BlockSpec((TB, 1), lambda j, p: (j, 0)),
            pl.BlockSpec((TB, DD), lambda j, p: (j, 0)),
            pl.BlockSpec((TB, 1, 1, D), lambda j, p: (j, p, 0, 0)),
            pl.BlockSpec((D, DD), lambda j, p: (0, 0)),
            pl.BlockSpec((1, DD), lambda j, p: (0, 0)),
        ],
        out_specs=pl.BlockSpec((TB, 2 * D), lambda j, p: (p * nj + j, 0)),
        out_shape=jax.ShapeDtypeStruct((P * B, 2 * D), jnp.float32),
    )(lv128, h2, data, noise4, W, b.reshape(1, DD))


def _gather_lv(mem128, d_idx, P, N, D, B):
    RPW = (B * P) // NW
    NCHUNK = RPW // 128

    def body(mem_ref, didx_ref, lv_ref, didx_v, gidx_v, rows_v, sem):
        wid = lax.axis_index("s") * NC + lax.axis_index("c")
        r0 = wid * RPW
        p = lax.shift_right_logical(wid, (B // RPW).bit_length() - 1)
        b0 = r0 - p * B
        pltpu.sync_copy(didx_ref.at[pl.ds(b0, RPW)], didx_v)
        off = p * (N // 2)

        def add_body(i, _):
            d = didx_v[pl.ds(i * L, L)]
            gidx_v[pl.ds(i * L, L)] = lax.shift_right_logical(d, 1) + off
            return 0

        lax.fori_loop(0, RPW // L, add_body, 0, unroll=4)
        descs = []
        for k in range(NCHUNK):
            descs.append(
                pltpu.async_copy(
                    mem_ref.at[gidx_v.at[pl.ds(k * 128, 128)]],
                    rows_v.at[pl.ds(k * 128, 128), :],
                    sem,
                )
            )
        for dsc in descs:
            dsc.wait()
        pltpu.sync_copy(rows_v, lv_ref.at[pl.ds(r0, RPW), :])

    mesh = plsc.VectorSubcoreMesh(core_axis_name="c", subcore_axis_name="s")
    return pl.kernel(
        body,
        out_type=jax.ShapeDtypeStruct((B * P, 2 * D), jnp.float32),
        mesh=mesh,
        compiler_params=pltpu.CompilerParams(needs_layout_passes=False),
        scratch_types=[
            pltpu.VMEM((RPW,), jnp.int32),
            pltpu.VMEM((RPW,), jnp.int32),
            pltpu.VMEM((RPW, 2 * D), jnp.float32),
            pltpu.SemaphoreType.DMA,
        ],
    )(mem128, d_idx)


def _scatter_copy(mem1, d_idx, upd128, P, N, D, B):
    CN = 49984 // NW
    TAIL = N - CN * NW
    SLAB = (CN + TAIL) * D
    MAXM = B + 2 * L
    NG = B // L

    def body(mem_ref, didx_ref, upd_ref, out_ref,
             didx_v, loc_v, b_v, ubid_v, upd_v, slab_v, sem, sem2):
        wid = lax.axis_index("s") * NC + lax.axis_index("c")
        lo = wid * CN
        is_w0 = wid == 0
        pltpu.sync_copy(didx_ref, didx_v)
        iota = lax.iota(jnp.int32, L)

        def fill_body(i, _):
            b_v[pl.ds(i * L, L)] = jnp.full((L,), wid, jnp.int32)
            return 0

        lax.fori_loop(0, MAXM // L, fill_body, 0, unroll=4)

        def scan_body(v, off):
            d = didx_v[pl.ds(v * L, L)]
            in_main = (d >= lo) & (d < lo + CN)
            in_tail = (d >= CN * NW) & is_w0
            m = in_main | in_tail
            locv = jnp.where(in_main, d - lo, d - CN * NW + CN)
            plsc.store_compressed(loc_v.at[pl.ds(off, L)], locv, mask=m)
            plsc.store_compressed(b_v.at[pl.ds(off, L)], iota + v * L, mask=m)
            return off + jnp.sum(m.astype(jnp.int32))

        cnt = lax.fori_loop(0, NG, scan_body, jnp.int32(0))
        nkb = lax.shift_right_logical(cnt + 127, 7)

        for p in range(P):
            src0 = (p * N + lo) * D
            in_desc = pltpu.async_copy(
                mem_ref.at[pl.ds(src0, CN * D)], slab_v.at[pl.ds(0, CN * D)], sem
            )
            in_desc.wait()

            @pl.when(is_w0)
            def _():
                pltpu.sync_copy(
                    mem_ref.at[pl.ds((p * N + CN * NW) * D, TAIL * D)],
                    slab_v.at[pl.ds(CN * D, TAIL * D)],
                )

            def batch_body(kb, _):
                def ub_body(i, _):
                    ubid_v[pl.ds(i * L, L)] = (
                        b_v[pl.ds(kb * 128 + i * L, L)] + p * B
                    )
                    return 0

                lax.fori_loop(0, 128 // L, ub_body, 0, unroll=4)
                pltpu.async_copy(
                    upd_ref.at[ubid_v], upd_v, sem2
                ).wait()

                def group_body(g, _):
                    base = kb * 128 + g * L
                    loc16 = loc_v[pl.ds(base, L)]
                    km = (base + iota) < cnt
                    woff = loc16 * D

                    def col_body(c, _):
                        vals = plsc.load_gather(
                            upd_v, [g * L + iota, jnp.full((L,), c, jnp.int32)]
                        )
                        plsc.addupdate_scatter(
                            slab_v, [woff + c], vals, mask=km
                        )
                        return 0

                    lax.fori_loop(0, D, col_body, 0, unroll=4)
                    return 0

                lax.fori_loop(0, 128 // L, group_body, 0)
                return 0

            lax.fori_loop(0, nkb, batch_body, 0)
            pltpu.sync_copy(
                slab_v.at[pl.ds(0, CN * D)], out_ref.at[pl.ds(src0, CN * D)]
            )

            @pl.when(is_w0)
            def _():
                pltpu.sync_copy(
                    slab_v.at[pl.ds(CN * D, TAIL * D)],
                    out_ref.at[pl.ds((p * N + CN * NW) * D, TAIL * D)],
                )

    mesh = plsc.VectorSubcoreMesh(core_axis_name="c", subcore_axis_name="s")
    return pl.kernel(
        body,
        out_type=jax.ShapeDtypeStruct((P * N * D,), jnp.float32),
        mesh=mesh,
        compiler_params=pltpu.CompilerParams(needs_layout_passes=False),
        scratch_types=[
            pltpu.VMEM((B,), jnp.int32),
            pltpu.VMEM((MAXM,), jnp.int32),
            pltpu.VMEM((MAXM,), jnp.int32),
            pltpu.VMEM((128,), jnp.int32),
            pltpu.VMEM((128, 2 * D), jnp.float32),
            pltpu.VMEM((SLAB,), jnp.float32),
            pltpu.SemaphoreType.DMA,
            pltpu.SemaphoreType.DMA,
        ],
    )(mem1, d_idx, upd128)


def kernel(mem, data, W, b, noise, d_idx):
    P, N, D = mem.shape
    B, DD = data.shape
    mem128 = mem.reshape(P * N // 2, 2 * D)
    mem1 = mem.reshape(P * N * D)
    lv128 = _gather_lv(mem128, d_idx, P, N, D, B)
    h2 = (d_idx % 2).astype(jnp.float32).reshape(B, 1)
    noise4 = noise.reshape(B, P, 1, D)
    upd128 = _compute_update(lv128, h2, data, noise4, W, b, P, B, D, DD)
    out1 = _scatter_copy(mem1, d_idx, upd128, P, N, D, B)
    return out1.reshape(P, N, D)


# speedup vs baseline: 2.5448x; 2.5448x over previous
import jax
import jax.numpy as jnp
from jax import lax
from jax.experimental import pallas as pl
from jax.experimental.pallas import tpu as pltpu
from jax.experimental.pallas import tpu_sc as plsc

LV_LR = 0.01
SIGMA = 1.0
NOISE_SCALE = (2.0 * LV_LR) ** 0.5

NC = 2
NS = 16
NW = NC * NS
L = 16
UB = 64


def _update_body(lv_ref, d_ref, nz_ref, w_ref, b_ref, out_ref):
    lv = lv_ref[:, :64]
    w = w_ref[...]
    pred = jnp.dot(lv, w, preferred_element_type=jnp.float32) + b_ref[...]
    resid = d_ref[...] - pred
    g = lax.dot_general(
        resid, w, (((1,), (1,)), ((), ())), preferred_element_type=jnp.float32
    ) - lv
    out_ref[:, :64] = LV_LR * g + NOISE_SCALE * nz_ref[:, 0, 0, :]
    out_ref[:, 64:] = jnp.zeros_like(g)


def _compute_update(lv128, data, noise4, W, b, P, B, D, DD):
    TB = 1024
    nj = B // TB
    return pl.pallas_call(
        _update_body,
        grid=(nj, P),
        in_specs=[
            pl.BlockSpec((TB, 2 * D), lambda j, p: (p * nj + j, 0)),
            pl.BlockSpec((TB, DD), lambda j, p: (j, 0)),
            pl.BlockSpec((TB, 1, 1, D), lambda j, p: (j, p, 0, 0)),
            pl.BlockSpec((D, DD), lambda j, p: (0, 0)),
            pl.BlockSpec((1, DD), lambda j, p: (0, 0)),
        ],
        out_specs=pl.BlockSpec((TB, 2 * D), lambda j, p: (p * nj + j, 0)),
        out_shape=jax.ShapeDtypeStruct((P * B, 2 * D), jnp.float32),
    )(lv128, data, noise4, W, b.reshape(1, DD))


def _scan_matches(didx_v, loc_v, b_v, lo, hi, NG):
    iota = lax.iota(jnp.int32, L)

    def scan_body(v, off):
        d = didx_v[pl.ds(v * L, L)]
        m = (d >= lo) & (d < hi)
        plsc.store_compressed(loc_v.at[pl.ds(off, L)], d - lo, mask=m)
        plsc.store_compressed(b_v.at[pl.ds(off, L)], iota + v * L, mask=m)
        return off + jnp.sum(m.astype(jnp.int32))

    return lax.fori_loop(0, NG, scan_body, jnp.int32(0))


def _prefill(b_v, wid, MAXM):
    def fill_body(i, _):
        b_v[pl.ds(i * L, L)] = jnp.full((L,), wid, jnp.int32)
        return 0

    lax.fori_loop(0, MAXM // L, fill_body, 0, unroll=4)


def _extract_lv(memT2, d_idx, P, N, D, B):
    CN = 1664
    LOMAX = ((N + 127) // 128) * 128 - CN
    MAXM = B + 2 * L
    NG = B // L
    PAD0 = P * B

    def body(mem_ref, didx_ref, lv_ref,
             didx_v, loc_v, b_v, ubid_v, stage_v, slab_v, sem, sem2):
        wid = lax.axis_index("s") * NC + lax.axis_index("c")
        lo = pl.multiple_of(jnp.minimum(wid * CN, LOMAX), 128)
        pltpu.sync_copy(didx_ref, didx_v)
        iota = lax.iota(jnp.int32, L)
        _prefill(b_v, wid, MAXM)
        cnt = _scan_matches(didx_v, loc_v, b_v, lo, lo + CN, NG)
        nkb = lax.shift_right_logical(cnt + (UB - 1), 6)

        for p in range(P):
            in_desc = pltpu.async_copy(
                mem_ref.at[pl.ds(p * D, D), pl.ds(lo, CN)],
                slab_v,
                sem,
            )
            in_desc.wait()

            def batch_body(kb, _):
                def ub_body(i, _):
                    pos = kb * UB + i * L
                    valid = (pos + iota) < cnt
                    b16 = b_v[pl.ds(pos, L)]
                    ubid_v[pl.ds(i * L, L)] = jnp.where(
                        valid, b16 + p * B, PAD0 + wid
                    )
                    return 0

                lax.fori_loop(0, UB // L, ub_body, 0, unroll=4)

                def group_body(g, _):
                    base = kb * UB + g * L
                    loc16 = loc_v[pl.ds(base, L)]
                    km = (base + iota) < cnt

                    def col_body(d, _):
                        dv = jnp.full((L,), d, jnp.int32)
                        vals = plsc.load_gather(slab_v, [dv, loc16], mask=km)
                        plsc.store_scatter(
                            stage_v, [g * L + iota, dv], vals, mask=km
                        )
                        return 0

                    lax.fori_loop(0, D, col_body, 0, unroll=4)
                    return 0

                lax.fori_loop(0, UB // L, group_body, 0)
                pltpu.async_copy(stage_v, lv_ref.at[ubid_v], sem2).wait()
                return 0

            lax.fori_loop(0, nkb, batch_body, 0)

    mesh = plsc.VectorSubcoreMesh(core_axis_name="c", subcore_axis_name="s")
    return pl.kernel(
        body,
        out_type=jax.ShapeDtypeStruct((P * B + 1024, 2 * D), jnp.float32),
        mesh=mesh,
        compiler_params=pltpu.CompilerParams(needs_layout_passes=False),
        scratch_types=[
            pltpu.VMEM((B,), jnp.int32),
            pltpu.VMEM((MAXM,), jnp.int32),
            pltpu.VMEM((MAXM,), jnp.int32),
            pltpu.VMEM((UB,), jnp.int32),
            pltpu.VMEM((UB, 2 * D), jnp.float32),
            pltpu.VMEM((D, CN), jnp.float32),
            pltpu.SemaphoreType.DMA,
            pltpu.SemaphoreType.DMA,
        ],
    )(memT2, d_idx)


def _scatter_copy(memT2, d_idx, upd128, P, N, D, B):
    CN = 1664
    LOMAX = ((N + 127) // 128) * 128 - CN
    MAXM = B + 2 * L
    NG = B // L

    def body(mem_ref, didx_ref, upd_ref, out_ref,
             didx_v, loc_v, b_v, ubid_v, upd_v, slab_v, sem, sem2):
        wid = lax.axis_index("s") * NC + lax.axis_index("c")
        lo = pl.multiple_of(jnp.minimum(wid * CN, LOMAX), 128)
        pltpu.sync_copy(didx_ref, didx_v)
        iota = lax.iota(jnp.int32, L)
        _prefill(b_v, wid, MAXM)
        cnt = _scan_matches(didx_v, loc_v, b_v, lo, lo + CN, NG)
        nkb = lax.shift_right_logical(cnt + (UB - 1), 6)

        for p in range(P):
            in_desc = pltpu.async_copy(
                mem_ref.at[pl.ds(p * D, D), pl.ds(lo, CN)],
                slab_v,
                sem,
            )
            in_desc.wait()

            def batch_body(kb, _):
                def ub_body(i, _):
                    ubid_v[pl.ds(i * L, L)] = (
                        b_v[pl.ds(kb * UB + i * L, L)] + p * B
                    )
                    return 0

                lax.fori_loop(0, UB // L, ub_body, 0, unroll=4)
                pltpu.async_copy(upd_ref.at[ubid_v], upd_v, sem2).wait()

                def group_body(g, _):
                    base = kb * UB + g * L
                    loc16 = loc_v[pl.ds(base, L)]
                    km = (base + iota) < cnt

                    def col_body(d, _):
                        dv = jnp.full((L,), d, jnp.int32)
                        vals = plsc.load_gather(
                            upd_v, [g * L + iota, dv], mask=km
                        )
                        plsc.addupdate_scatter(
                            slab_v, [dv, loc16], vals, mask=km
                        )
                        return 0

                    lax.fori_loop(0, D, col_body, 0, unroll=4)
                    return 0

                lax.fori_loop(0, UB // L, group_body, 0)
                return 0

            lax.fori_loop(0, nkb, batch_body, 0)
            pltpu.sync_copy(
                slab_v,
                out_ref.at[pl.ds(p * D, D), pl.ds(lo, CN)],
            )

    mesh = plsc.VectorSubcoreMesh(core_axis_name="c", subcore_axis_name="s")
    return pl.kernel(
        body,
        out_type=jax.ShapeDtypeStruct((P * D, N), jnp.float32),
        mesh=mesh,
        compiler_params=pltpu.CompilerParams(needs_layout_passes=False),
        scratch_types=[
            pltpu.VMEM((B,), jnp.int32),
            pltpu.VMEM((MAXM,), jnp.int32),
            pltpu.VMEM((MAXM,), jnp.int32),
            pltpu.VMEM((UB,), jnp.int32),
            pltpu.VMEM((UB, 2 * D), jnp.float32),
            pltpu.VMEM((D, CN), jnp.float32),
            pltpu.SemaphoreType.DMA,
            pltpu.SemaphoreType.DMA,
        ],
    )(memT2, d_idx, upd128)


def kernel(mem, data, W, b, noise, d_idx):
    P, N, D = mem.shape
    B, DD = data.shape
    memT2 = jnp.transpose(mem, (0, 2, 1)).reshape(P * D, N)
    lv128 = _extract_lv(memT2, d_idx, P, N, D, B)
    noise4 = noise.reshape(B, P, 1, D)
    upd128 = _compute_update(lv128, data, noise4, W, b, P, B, D, DD)
    outT2 = _scatter_copy(memT2, d_idx, upd128, P, N, D, B)
    return outT2.reshape(P, D, N).transpose(0, 2, 1)
